# baseline (speedup 1.0000x reference)
import functools

import jax
import jax.numpy as jnp
from jax.experimental import pallas as pl
from jax.experimental.pallas import tpu as pltpu

K = 16
BIG = 3.0e38
PAD_COORD = 1.0e5


def _proj_body(x_ref, ws_ref, bs_ref, wh_ref, bh_ref, out_ref):
    x = x_ref[...]
    s = jnp.dot(x, ws_ref[...], preferred_element_type=jnp.float32) + bs_ref[...][None, :]
    h = jnp.dot(x, wh_ref[...], preferred_element_type=jnp.float32) + bh_ref[...][None, :]
    s2 = jnp.sum(s * s, axis=1, keepdims=True)
    out_ref[...] = jnp.concatenate([s, h, s2], axis=1)


def _knn_body(num_tiles, t_size, ct_ref, c2_ref, h_ref, q_ref, q2_ref,
              feat_ref, masked_ref, best_ref, thr_ref, acc_sum_ref,
              acc_max_ref, cont_ref):
    qn = q_ref.shape[0]
    q = q_ref[...]
    q2 = q2_ref[...]

    def tile_d2(t):
        c = ct_ref[:, pl.ds(t * t_size, t_size)]
        c2 = c2_ref[:, pl.ds(t * t_size, t_size)]
        dot = jnp.dot(q, c, preferred_element_type=jnp.float32)
        return q2 + c2 - 2.0 * dot

    def insert(v, take):
        b = best_ref[...]
        bm = jnp.max(b, axis=1, keepdims=True)
        nb = jnp.where((b == bm) & take, v, b)
        best_ref[...] = nb
        thr_ref[...] = jnp.max(nb, axis=1, keepdims=True)
        cont_ref[0] = jnp.max(take.astype(jnp.int32))

    lane = jax.lax.broadcasted_iota(jnp.int32, (qn, t_size), 1)

    def tile_work(d2):
        v1 = jnp.min(d2, axis=1, keepdims=True)
        need1 = v1 < thr_ref[...]

        @pl.when(jnp.max(need1.astype(jnp.int32)) == 1)
        def _tile_work():
            masked_ref[...] = jnp.where(d2 < thr_ref[...], d2, BIG)
            insert(v1, need1)
            masked_ref[...] = jnp.where(masked_ref[...] == v1, BIG,
                                        masked_ref[...])
            for _ in range(K - 1):
                @pl.when(cont_ref[0] == 1)
                def _extract():
                    m = masked_ref[...]
                    v = jnp.min(m, axis=1, keepdims=True)
                    insert(v, v < thr_ref[...])
                    masked_ref[...] = jnp.where(m == v, BIG, m)

    d2_0 = tile_d2(0)
    seed = d2_0[:, :K]
    best_ref[...] = seed
    thr_ref[...] = jnp.max(seed, axis=1, keepdims=True)
    tile_work(jnp.where(lane < K, BIG, d2_0))

    def pass1(t, carry):
        tile_work(tile_d2(t))
        return carry

    jax.lax.fori_loop(1, num_tiles, pass1, 0)

    thr = thr_ref[...]
    acc_sum_ref[...] = jnp.zeros((qn, 1), jnp.float32)
    acc_max_ref[...] = jnp.full((qn, 1), -BIG, jnp.float32)

    def pass2(t, carry):
        d2 = tile_d2(t)
        sel = d2 <= thr
        d2e = jnp.zeros_like(d2)
        for d in range(3):
            diff = q[:, d:d + 1] - ct_ref[d:d + 1, pl.ds(t * t_size, t_size)]
            d2e = d2e + diff * diff
        h = h_ref[:, pl.ds(t * t_size, t_size)]
        msg = h * jnp.exp(-10.0 * d2e)
        acc_sum_ref[...] += jnp.sum(jnp.where(sel, msg, 0.0),
                                    axis=1, keepdims=True)
        acc_max_ref[...] = jnp.maximum(
            acc_max_ref[...],
            jnp.max(jnp.where(sel, msg, -BIG), axis=1, keepdims=True))
        return carry

    jax.lax.fori_loop(0, num_tiles, pass2, 0)
    feat_ref[...] = jnp.concatenate(
        [acc_sum_ref[...] * (1.0 / K), acc_max_ref[...]], axis=1)


def _head_body(x_ref, feat_ref, wo1_ref, wo2_ref, bo2_ref, out_ref):
    conv = (
        jnp.dot(x_ref[...], wo1_ref[...], preferred_element_type=jnp.float32)
        + jnp.dot(feat_ref[...], wo2_ref[...], preferred_element_type=jnp.float32)
        + bo2_ref[...][None, :]
    )
    t = jnp.where(conv > 0, conv, jnp.exp(jnp.minimum(conv, 0.0)) - 1.0)
    part = jnp.sum(t, axis=0, keepdims=True)

    @pl.when(pl.program_id(0) == 0)
    def _init():
        out_ref[...] = jnp.zeros_like(out_ref)

    out_ref[...] += part


def _gravnet_feat(sh, n, q_blk, t_size):
    npad = -(-n // max(q_blk, t_size)) * max(q_blk, t_size)
    npad = -(-npad // t_size) * t_size
    npad = -(-npad // q_blk) * q_blk
    pad_rows = npad - n
    pad = jnp.concatenate(
        [jnp.full((pad_rows, 3), PAD_COORD, jnp.float32),
         jnp.zeros((pad_rows, 1), jnp.float32),
         jnp.full((pad_rows, 1), 3.0 * PAD_COORD * PAD_COORD, jnp.float32)],
        axis=1)
    shp = jnp.concatenate([sh, pad], axis=0)
    ct = shp[:, :3].T
    c2 = shp[:, 4:5].T
    hrow = shp[:, 3:4].T
    num_tiles = npad // t_size

    body = functools.partial(_knn_body, num_tiles, t_size)
    feat = pl.pallas_call(
        body,
        grid=(npad // q_blk,),
        in_specs=[
            pl.BlockSpec((3, npad), lambda i: (0, 0)),
            pl.BlockSpec((1, npad), lambda i: (0, 0)),
            pl.BlockSpec((1, npad), lambda i: (0, 0)),
            pl.BlockSpec((q_blk, 3), lambda i: (i, 0)),
            pl.BlockSpec((q_blk, 1), lambda i: (i, 0)),
        ],
        out_specs=pl.BlockSpec((q_blk, 2), lambda i: (i, 0)),
        out_shape=jax.ShapeDtypeStruct((npad, 2), jnp.float32),
        scratch_shapes=[
            pltpu.VMEM((q_blk, t_size), jnp.float32),
            pltpu.VMEM((q_blk, K), jnp.float32),
            pltpu.VMEM((q_blk, 1), jnp.float32),
            pltpu.VMEM((q_blk, 1), jnp.float32),
            pltpu.VMEM((q_blk, 1), jnp.float32),
            pltpu.SMEM((1,), jnp.int32),
        ],
        compiler_params=pltpu.CompilerParams(
            dimension_semantics=("arbitrary",)),
    )(ct, c2, hrow, shp[:, :3], shp[:, 4:5])
    return feat[:n]


def kernel(x, batch, W_s, b_s, W_h, b_h, W_o1, W_o2, b_o2, W_out, b_out):
    n = x.shape[0]
    blk = 2000

    sh = pl.pallas_call(
        _proj_body,
        grid=(n // blk,),
        in_specs=[
            pl.BlockSpec((blk, 4), lambda i: (i, 0)),
            pl.BlockSpec((4, 3), lambda i: (0, 0)),
            pl.BlockSpec((3,), lambda i: (0,)),
            pl.BlockSpec((4, 1), lambda i: (0, 0)),
            pl.BlockSpec((1,), lambda i: (0,)),
        ],
        out_specs=pl.BlockSpec((blk, 5), lambda i: (i, 0)),
        out_shape=jax.ShapeDtypeStruct((n, 5), jnp.float32),
    )(x, W_s, b_s, W_h, b_h)

    feat = _gravnet_feat(sh, n, 512, 4096)

    pooled = pl.pallas_call(
        _head_body,
        grid=(n // blk,),
        in_specs=[
            pl.BlockSpec((blk, 4), lambda i: (i, 0)),
            pl.BlockSpec((blk, 2), lambda i: (i, 0)),
            pl.BlockSpec((4, 64), lambda i: (0, 0)),
            pl.BlockSpec((2, 64), lambda i: (0, 0)),
            pl.BlockSpec((64,), lambda i: (0,)),
        ],
        out_specs=pl.BlockSpec((1, 64), lambda i: (0, 0)),
        out_shape=jax.ShapeDtypeStruct((1, 64), jnp.float32),
    )(x, feat, W_o1, W_o2, b_o2)

    return pooled @ W_out + b_out


# speedup vs baseline: 1.1357x; 1.1357x over previous
import functools

import jax
import jax.numpy as jnp
from jax.experimental import pallas as pl
from jax.experimental.pallas import tpu as pltpu

K = 16
BIG = 3.0e38
PAD_COORD = 1.0e5


def _proj_body(x_ref, ws_ref, bs_ref, wh_ref, bh_ref, out_ref):
    x = x_ref[...]
    s = jnp.dot(x, ws_ref[...], preferred_element_type=jnp.float32) + bs_ref[...][None, :]
    h = jnp.dot(x, wh_ref[...], preferred_element_type=jnp.float32) + bh_ref[...][None, :]
    s2 = jnp.sum(s * s, axis=1, keepdims=True)
    out_ref[...] = jnp.concatenate([s, h, s2], axis=1)


def _knn_body(num_tiles, t_size, ct_ref, c2_ref, h_ref, q_ref, q2_ref,
              feat_ref, masked_ref, best_ref, thr_ref, acc_sum_ref,
              acc_max_ref, cont_ref):
    qn = q_ref.shape[0]
    q = q_ref[...]
    q2 = q2_ref[...]

    def tile_d2(t):
        c = ct_ref[:, pl.ds(t * t_size, t_size)]
        c2 = c2_ref[:, pl.ds(t * t_size, t_size)]
        dot = jnp.dot(q, c, preferred_element_type=jnp.float32)
        return q2 + c2 - 2.0 * dot

    def insert(v, take):
        b = best_ref[...]
        bm = jnp.max(b, axis=1, keepdims=True)
        nb = jnp.where((b == bm) & take, v, b)
        best_ref[...] = nb
        thr_ref[...] = jnp.max(nb, axis=1, keepdims=True)
        cont_ref[0] = jnp.max(take.astype(jnp.int32))

    lane = jax.lax.broadcasted_iota(jnp.int32, (qn, t_size), 1)

    def tile_work(d2):
        v1 = jnp.min(d2, axis=1, keepdims=True)
        need1 = v1 < thr_ref[...]

        @pl.when(jnp.max(need1.astype(jnp.int32)) == 1)
        def _tile_work():
            masked_ref[...] = jnp.where(d2 < thr_ref[...], d2, BIG)
            insert(v1, need1)
            masked_ref[...] = jnp.where(masked_ref[...] == v1, BIG,
                                        masked_ref[...])
            for _ in range(K - 1):
                @pl.when(cont_ref[0] == 1)
                def _extract():
                    m = masked_ref[...]
                    v = jnp.min(m, axis=1, keepdims=True)
                    insert(v, v < thr_ref[...])
                    masked_ref[...] = jnp.where(m == v, BIG, m)

    d2_0 = tile_d2(0)
    seed = d2_0[:, :K]
    best_ref[...] = seed
    thr_ref[...] = jnp.max(seed, axis=1, keepdims=True)
    tile_work(jnp.where(lane < K, BIG, d2_0))

    def pass1(t, carry):
        tile_work(tile_d2(t))
        return carry

    jax.lax.fori_loop(1, num_tiles, pass1, 0)

    thr = thr_ref[...]
    acc_sum_ref[...] = jnp.zeros((qn, 1), jnp.float32)
    acc_max_ref[...] = jnp.full((qn, 1), -BIG, jnp.float32)

    def pass2(t, carry):
        d2 = tile_d2(t)
        sel = d2 <= thr
        d2e = jnp.zeros_like(d2)
        for d in range(3):
            diff = q[:, d:d + 1] - ct_ref[d:d + 1, pl.ds(t * t_size, t_size)]
            d2e = d2e + diff * diff
        h = h_ref[:, pl.ds(t * t_size, t_size)]
        msg = h * jnp.exp(-10.0 * d2e)
        acc_sum_ref[...] += jnp.sum(jnp.where(sel, msg, 0.0),
                                    axis=1, keepdims=True)
        acc_max_ref[...] = jnp.maximum(
            acc_max_ref[...],
            jnp.max(jnp.where(sel, msg, -BIG), axis=1, keepdims=True))
        return carry

    jax.lax.fori_loop(0, num_tiles, pass2, 0)
    feat_ref[...] = jnp.concatenate(
        [acc_sum_ref[...] * (1.0 / K), acc_max_ref[...]], axis=1)


def _head_body(x_ref, feat_ref, wo1_ref, wo2_ref, bo2_ref, out_ref):
    conv = (
        jnp.dot(x_ref[...], wo1_ref[...], preferred_element_type=jnp.float32)
        + jnp.dot(feat_ref[...], wo2_ref[...], preferred_element_type=jnp.float32)
        + bo2_ref[...][None, :]
    )
    t = jnp.where(conv > 0, conv, jnp.exp(jnp.minimum(conv, 0.0)) - 1.0)
    part = jnp.sum(t, axis=0, keepdims=True)

    @pl.when(pl.program_id(0) == 0)
    def _init():
        out_ref[...] = jnp.zeros_like(out_ref)

    out_ref[...] += part


def _gravnet_feat(sh, n, q_blk, t_size):
    npad = -(-n // max(q_blk, t_size)) * max(q_blk, t_size)
    npad = -(-npad // t_size) * t_size
    npad = -(-npad // q_blk) * q_blk
    pad_rows = npad - n
    pad = jnp.concatenate(
        [jnp.full((pad_rows, 3), PAD_COORD, jnp.float32),
         jnp.zeros((pad_rows, 1), jnp.float32),
         jnp.full((pad_rows, 1), 3.0 * PAD_COORD * PAD_COORD, jnp.float32)],
        axis=1)
    shp = jnp.concatenate([sh, pad], axis=0)
    ct = shp[:, :3].T
    c2 = shp[:, 4:5].T
    hrow = shp[:, 3:4].T
    num_tiles = npad // t_size

    body = functools.partial(_knn_body, num_tiles, t_size)
    feat = pl.pallas_call(
        body,
        grid=(npad // q_blk,),
        in_specs=[
            pl.BlockSpec((3, npad), lambda i: (0, 0)),
            pl.BlockSpec((1, npad), lambda i: (0, 0)),
            pl.BlockSpec((1, npad), lambda i: (0, 0)),
            pl.BlockSpec((q_blk, 3), lambda i: (i, 0)),
            pl.BlockSpec((q_blk, 1), lambda i: (i, 0)),
        ],
        out_specs=pl.BlockSpec((q_blk, 2), lambda i: (i, 0)),
        out_shape=jax.ShapeDtypeStruct((npad, 2), jnp.float32),
        scratch_shapes=[
            pltpu.VMEM((q_blk, t_size), jnp.float32),
            pltpu.VMEM((q_blk, K), jnp.float32),
            pltpu.VMEM((q_blk, 1), jnp.float32),
            pltpu.VMEM((q_blk, 1), jnp.float32),
            pltpu.VMEM((q_blk, 1), jnp.float32),
            pltpu.SMEM((1,), jnp.int32),
        ],
        compiler_params=pltpu.CompilerParams(
            dimension_semantics=("arbitrary",)),
    )(ct, c2, hrow, shp[:, :3], shp[:, 4:5])
    return feat[:n]


def kernel(x, batch, W_s, b_s, W_h, b_h, W_o1, W_o2, b_o2, W_out, b_out):
    n = x.shape[0]
    blk = 2000

    sh = pl.pallas_call(
        _proj_body,
        grid=(n // blk,),
        in_specs=[
            pl.BlockSpec((blk, 4), lambda i: (i, 0)),
            pl.BlockSpec((4, 3), lambda i: (0, 0)),
            pl.BlockSpec((3,), lambda i: (0,)),
            pl.BlockSpec((4, 1), lambda i: (0, 0)),
            pl.BlockSpec((1,), lambda i: (0,)),
        ],
        out_specs=pl.BlockSpec((blk, 5), lambda i: (i, 0)),
        out_shape=jax.ShapeDtypeStruct((n, 5), jnp.float32),
    )(x, W_s, b_s, W_h, b_h)

    feat = _gravnet_feat(sh, n, 1024, 2048)

    pooled = pl.pallas_call(
        _head_body,
        grid=(n // blk,),
        in_specs=[
            pl.BlockSpec((blk, 4), lambda i: (i, 0)),
            pl.BlockSpec((blk, 2), lambda i: (i, 0)),
            pl.BlockSpec((4, 64), lambda i: (0, 0)),
            pl.BlockSpec((2, 64), lambda i: (0, 0)),
            pl.BlockSpec((64,), lambda i: (0,)),
        ],
        out_specs=pl.BlockSpec((1, 64), lambda i: (0, 0)),
        out_shape=jax.ShapeDtypeStruct((1, 64), jnp.float32),
    )(x, feat, W_o1, W_o2, b_o2)

    return pooled @ W_out + b_out
